# baseline (speedup 1.0000x reference)
import functools

import jax
import jax.numpy as jnp
from jax import lax
from jax.experimental import pallas as pl
from jax.experimental.pallas import tpu as pltpu
from jax.experimental.pallas import tpu_sc as plsc

_CHUNKS = (64, 64, 128, 256)


def _make_gather(B: int, V: int, D: int):
    info = plsc.get_sparse_core_info()
    nw = info.num_cores * info.num_subcores
    b_per_w = B // nw
    assert b_per_w == sum(_CHUNKS)
    offs = [sum(_CHUNKS[:c]) for c in range(len(_CHUNKS))]
    stage_rows = 128
    full_tiles = (V - 1) // stage_rows
    tail_rows = V - full_tiles * stage_rows

    mesh = plsc.VectorSubcoreMesh(core_axis_name="c", subcore_axis_name="s")

    @functools.partial(
        pl.kernel,
        mesh=mesh,
        out_type=jax.ShapeDtypeStruct((B, D), jnp.float32),
        scratch_types=[
            pltpu.VMEM_SHARED((V, D), jnp.float32),
            pltpu.VMEM((b_per_w,), jnp.int32),
            pltpu.VMEM((b_per_w, D), jnp.float32),
        ]
        + [pltpu.SemaphoreType.DMA] * len(_CHUNKS)
        + [pltpu.SemaphoreType.DMA],
    )
    def gather_kernel(idx_hbm, table_hbm, out_hbm, table_sp, idx_v, rows_v, *sems):
        gsems, ssem = sems[: len(_CHUNKS)], sems[len(_CHUNKS)]
        sid = lax.axis_index("s")
        wid = sid * info.num_cores + lax.axis_index("c")
        base = wid * b_per_w
        @pl.when(sid < full_tiles)
        def _():
            row0 = sid * stage_rows
            pltpu.sync_copy(
                table_hbm.at[pl.ds(row0, stage_rows)],
                table_sp.at[pl.ds(row0, stage_rows)],
            )

        @pl.when(sid == full_tiles)
        def _():
            row0 = full_tiles * stage_rows
            pltpu.sync_copy(
                table_hbm.at[pl.ds(row0, tail_rows)],
                table_sp.at[pl.ds(row0, tail_rows)],
            )

        pltpu.sync_copy(idx_hbm.at[pl.ds(base, b_per_w)], idx_v)
        plsc.subcore_barrier()
        gathers = [
            pltpu.async_copy(
                table_sp.at[idx_v.at[pl.ds(offs[c], n)]],
                rows_v.at[pl.ds(offs[c], n)],
                gsems[c],
            )
            for c, n in enumerate(_CHUNKS)
        ]
        stores = []
        for c, n in enumerate(_CHUNKS):
            gathers[c].wait()
            stores.append(
                pltpu.async_copy(
                    rows_v.at[pl.ds(offs[c], n)],
                    out_hbm.at[pl.ds(base + offs[c], n)],
                    ssem,
                )
            )
        for s in stores:
            s.wait()

    return gather_kernel


@jax.jit
def kernel(condition, embed_table):
    B, = condition.shape
    V, D = embed_table.shape
    return _make_gather(B, V, D)(condition.astype(jnp.int32), embed_table)


# speedup vs baseline: 1.0034x; 1.0034x over previous
import functools

import jax
import jax.numpy as jnp
from jax import lax
from jax.experimental import pallas as pl
from jax.experimental.pallas import tpu as pltpu
from jax.experimental.pallas import tpu_sc as plsc

_NCHUNKS = 4


def _make_gather(B: int, V: int, D: int):
    info = plsc.get_sparse_core_info()
    nw = info.num_cores * info.num_subcores
    assert B % (nw * _NCHUNKS) == 0
    b_per_w = B // nw
    chunk = b_per_w // _NCHUNKS

    mesh = plsc.VectorSubcoreMesh(core_axis_name="c", subcore_axis_name="s")

    @functools.partial(
        pl.kernel,
        mesh=mesh,
        out_type=jax.ShapeDtypeStruct((B, D), jnp.float32),
        scratch_types=[
            pltpu.VMEM_SHARED((V, D), jnp.float32),
            pltpu.VMEM((_NCHUNKS, chunk), jnp.int32),
            pltpu.VMEM((_NCHUNKS, chunk, D), jnp.float32),
        ]
        + [pltpu.SemaphoreType.DMA] * _NCHUNKS
        + [pltpu.SemaphoreType.DMA],
    )
    def gather_kernel(idx_hbm, table_hbm, out_hbm, table_sp, idx_v, rows_v, *sems):
        gsems, ssem = sems[:_NCHUNKS], sems[_NCHUNKS]
        sid = lax.axis_index("s")
        wid = sid * info.num_cores + lax.axis_index("c")
        base = wid * b_per_w
        @pl.when(sid == 0)
        def _():
            pltpu.sync_copy(table_hbm, table_sp)

        pltpu.sync_copy(idx_hbm.at[wid], idx_v)
        plsc.subcore_barrier()
        gathers = [
            pltpu.async_copy(table_sp.at[idx_v.at[c]], rows_v.at[c], gsems[c])
            for c in range(_NCHUNKS)
        ]
        stores = []
        for c in range(_NCHUNKS):
            gathers[c].wait()
            stores.append(
                pltpu.async_copy(
                    rows_v.at[c], out_hbm.at[pl.ds(base + c * chunk, chunk)], ssem
                )
            )
        for s in stores:
            s.wait()

    return gather_kernel


@jax.jit
def kernel(condition, embed_table):
    B, = condition.shape
    V, D = embed_table.shape
    info = plsc.get_sparse_core_info()
    nw = info.num_cores * info.num_subcores
    idx = condition.astype(jnp.int32).reshape(nw, _NCHUNKS, B // (nw * _NCHUNKS))
    return _make_gather(B, V, D)(idx, embed_table)


# speedup vs baseline: 1.0103x; 1.0069x over previous
import functools

import jax
import jax.numpy as jnp
from jax import lax
from jax.experimental import pallas as pl
from jax.experimental.pallas import tpu as pltpu
from jax.experimental.pallas import tpu_sc as plsc

_CHUNKS = (64, 128, 128, 192)


def _make_gather(B: int, V: int, D: int):
    info = plsc.get_sparse_core_info()
    nw = info.num_cores * info.num_subcores
    b_per_w = B // nw
    assert b_per_w == sum(_CHUNKS)
    offs = [sum(_CHUNKS[:c]) for c in range(len(_CHUNKS))]
    stage_split = 512

    mesh = plsc.VectorSubcoreMesh(core_axis_name="c", subcore_axis_name="s")

    @functools.partial(
        pl.kernel,
        mesh=mesh,
        out_type=jax.ShapeDtypeStruct((B, D), jnp.float32),
        scratch_types=[
            pltpu.VMEM_SHARED((V, D), jnp.float32),
            pltpu.VMEM((b_per_w,), jnp.int32),
            pltpu.VMEM((b_per_w, D), jnp.float32),
        ]
        + [pltpu.SemaphoreType.DMA] * len(_CHUNKS)
        + [pltpu.SemaphoreType.DMA],
    )
    def gather_kernel(idx_hbm, table_hbm, out_hbm, table_sp, idx_v, rows_v, *sems):
        gsems, ssem = sems[: len(_CHUNKS)], sems[len(_CHUNKS)]
        sid = lax.axis_index("s")
        wid = sid * info.num_cores + lax.axis_index("c")
        base = wid * b_per_w
        pltpu.sync_copy(idx_hbm.at[pl.ds(base, b_per_w)], idx_v)
        g0 = pltpu.async_copy(
            table_hbm.at[idx_v.at[pl.ds(0, _CHUNKS[0])]],
            rows_v.at[pl.ds(0, _CHUNKS[0])],
            gsems[0],
        )

        @pl.when(sid == 0)
        def _():
            pltpu.sync_copy(
                table_hbm.at[pl.ds(0, stage_split)],
                table_sp.at[pl.ds(0, stage_split)],
            )

        @pl.when(sid == 1)
        def _():
            pltpu.sync_copy(
                table_hbm.at[pl.ds(stage_split, V - stage_split)],
                table_sp.at[pl.ds(stage_split, V - stage_split)],
            )

        g0.wait()
        stores = [
            pltpu.async_copy(
                rows_v.at[pl.ds(0, _CHUNKS[0])],
                out_hbm.at[pl.ds(base, _CHUNKS[0])],
                ssem,
            )
        ]
        plsc.subcore_barrier()
        gathers = [
            pltpu.async_copy(
                table_sp.at[idx_v.at[pl.ds(offs[c], n)]],
                rows_v.at[pl.ds(offs[c], n)],
                gsems[c],
            )
            for c, n in enumerate(_CHUNKS)
            if c > 0
        ]
        for i, (c, n) in enumerate([(c, n) for c, n in enumerate(_CHUNKS) if c > 0]):
            gathers[i].wait()
            stores.append(
                pltpu.async_copy(
                    rows_v.at[pl.ds(offs[c], n)],
                    out_hbm.at[pl.ds(base + offs[c], n)],
                    ssem,
                )
            )
        for s in stores:
            s.wait()

    return gather_kernel


@jax.jit
def kernel(condition, embed_table):
    B, = condition.shape
    V, D = embed_table.shape
    return _make_gather(B, V, D)(condition.astype(jnp.int32), embed_table)
